# baseline (speedup 1.0000x reference)
import jax
import jax.numpy as jnp
from jax import lax
from jax.experimental import pallas as pl
from jax.experimental.pallas import tpu as pltpu
from jax.experimental.pallas import tpu_sc as plsc

N = 10000
E = 320000
D_IN = 128
D_H = 64
CHUNK = 128
NCHUNKS = E // CHUNK
NC = 2
NS = 16
NW = NC * NS
NPAD = 10240
ROWS_PER_TILE = NPAD // NS
CPT = NCHUNKS // NW
CPT_REM = NCHUNKS % NW
MAXCPT = CPT + 1
NBUF = 8
TOUT = (MAXCPT + NBUF - 1) // NBUF
GB = 10
BN = N // GB

_mesh = plsc.VectorSubcoreMesh(
    core_axis_name="c", subcore_axis_name="s", num_cores=NC, num_subcores=NS
)
_sc_params = pltpu.CompilerParams(use_tc_tiling_on_sc=False)


def _leaky(v):
    return jnp.where(v >= 0, v, 0.01 * v)


def _tile_chunk_range(w):
    count = jnp.where(w < CPT_REM, CPT + 1, CPT)
    start = CPT * w + jnp.minimum(w, CPT_REM)
    return start, count


def _preload_idx(ei, row, start, count, idx_all):
    base = pl.multiple_of(start * CHUNK, CHUNK)
    pltpu.sync_copy(ei.at[row, pl.ds(base, CPT * CHUNK)],
                    idx_all.at[pl.ds(0, CPT * CHUNK)])

    @pl.when(count > CPT)
    def _():
        base2 = pl.multiple_of((start + CPT) * CHUNK, CHUNK)
        pltpu.sync_copy(ei.at[row, pl.ds(base2, CHUNK)],
                        idx_all.at[pl.ds(CPT * CHUNK, CHUNK)])


def _chunk_idx(idx_all, j):
    return idx_all.at[pl.ds(pl.multiple_of(j * CHUNK, CHUNK), CHUNK)]


DEG_W = 16


def _sc_deg_body(ei, zeros_hbm, ones_hbm, out, didx_all, ones_v, zbuf,
                 acc, ssem):
    c = lax.axis_index("c")
    s = lax.axis_index("s")
    w = s * NC + c
    start, count = _tile_chunk_range(w)

    pltpu.sync_copy(ones_hbm, ones_v)
    pltpu.sync_copy(zeros_hbm, zbuf)
    pltpu.sync_copy(zbuf, acc.at[pl.ds(s * ROWS_PER_TILE, ROWS_PER_TILE)])
    _preload_idx(ei, 1, start, count, didx_all)
    plsc.subcore_barrier()

    def obody(k, carry):
        base = k * NBUF
        for b in range(NBUF):
            j = base + b

            @pl.when(jnp.logical_and(k > 0, j < count))
            def _(b=b, j=j):
                pltpu.make_async_copy(
                    ones_v, acc.at[_chunk_idx(didx_all, j - NBUF)], ssem.at[b]
                ).wait()

            @pl.when(j < count)
            def _(b=b, j=j):
                pltpu.async_copy(
                    ones_v, acc.at[_chunk_idx(didx_all, j)], ssem.at[b],
                    add=True,
                )

        return carry

    lax.fori_loop(0, TOUT, obody, 0)
    for b in range(NBUF):
        pltpu.make_async_copy(
            ones_v, acc.at[_chunk_idx(didx_all, b)], ssem.at[b]
        ).wait()
    plsc.subcore_barrier()
    pltpu.sync_copy(
        acc.at[pl.ds(s * ROWS_PER_TILE, ROWS_PER_TILE)],
        out.at[c, pl.ds(s * ROWS_PER_TILE, ROWS_PER_TILE)],
    )


_sc_deg = pl.kernel(
    _sc_deg_body,
    out_type=jax.ShapeDtypeStruct((NC, NPAD, DEG_W), jnp.float32),
    mesh=_mesh,
    compiler_params=_sc_params,
    scratch_types=[
        pltpu.VMEM((MAXCPT * CHUNK,), jnp.int32),
        pltpu.VMEM((CHUNK, DEG_W), jnp.float32),
        pltpu.VMEM((ROWS_PER_TILE, DEG_W), jnp.float32),
        pltpu.VMEM_SHARED((NPAD, DEG_W), jnp.float32),
        pltpu.SemaphoreType.DMA((NBUF,)),
    ],
)


def _sc_scat_body(hp, ei, zeros_hbm, out, sidx_all, didx_all, rows,
                  acc, gsem, ssem):
    c = lax.axis_index("c")
    s = lax.axis_index("s")
    w = s * NC + c
    start, count = _tile_chunk_range(w)

    pltpu.sync_copy(zeros_hbm, rows.at[0])
    for i in range(ROWS_PER_TILE // CHUNK):
        pltpu.sync_copy(
            rows.at[0], acc.at[pl.ds(s * ROWS_PER_TILE + i * CHUNK, CHUNK)]
        )
    _preload_idx(ei, 0, start, count, sidx_all)
    _preload_idx(ei, 1, start, count, didx_all)
    plsc.subcore_barrier()

    def obody(k, carry):
        base = k * NBUF
        for b in range(NBUF):
            j = base + b

            @pl.when(jnp.logical_and(k > 0, j < count))
            def _(b=b, j=j):
                pltpu.make_async_copy(
                    rows.at[b], acc.at[_chunk_idx(didx_all, j - NBUF)],
                    ssem.at[b],
                ).wait()

            @pl.when(j < count)
            def _(b=b, j=j):
                pltpu.async_copy(
                    hp.at[_chunk_idx(sidx_all, j)], rows.at[b], gsem.at[b]
                )

        for b in range(NBUF):
            j = base + b

            @pl.when(j < count)
            def _(b=b, j=j):
                pltpu.make_async_copy(
                    hp.at[_chunk_idx(sidx_all, j)], rows.at[b], gsem.at[b]
                ).wait()
                pltpu.async_copy(
                    rows.at[b], acc.at[_chunk_idx(didx_all, j)], ssem.at[b],
                    add=True,
                )

        return carry

    lax.fori_loop(0, TOUT, obody, 0)
    for b in range(NBUF):
        pltpu.make_async_copy(
            rows.at[b], acc.at[_chunk_idx(didx_all, b)], ssem.at[b]
        ).wait()
    plsc.subcore_barrier()
    pltpu.sync_copy(
        acc.at[pl.ds(s * ROWS_PER_TILE, ROWS_PER_TILE)],
        out.at[c, pl.ds(s * ROWS_PER_TILE, ROWS_PER_TILE)],
    )


_sc_scat = pl.kernel(
    _sc_scat_body,
    out_type=jax.ShapeDtypeStruct((NC, NPAD, D_H), jnp.bfloat16),
    mesh=_mesh,
    compiler_params=_sc_params,
    scratch_types=[
        pltpu.VMEM((MAXCPT * CHUNK,), jnp.int32),
        pltpu.VMEM((MAXCPT * CHUNK,), jnp.int32),
        pltpu.VMEM((NBUF, CHUNK, D_H), jnp.bfloat16),
        pltpu.VMEM_SHARED((NPAD, D_H), jnp.bfloat16),
        pltpu.SemaphoreType.DMA((NBUF,)),
        pltpu.SemaphoreType.DMA((NBUF,)),
    ],
)


def _tc_first_body(degp_ref, x_ref, w1_ref, hp_ref, dinv_ref):
    deg = degp_ref[0, :, 0] + degp_ref[1, :, 0] + 1.0
    dinv = lax.rsqrt(deg).reshape(BN, 1)
    h = jnp.dot(x_ref[...], w1_ref[...], preferred_element_type=jnp.float32)
    hp_ref[...] = (h * dinv).astype(jnp.bfloat16)
    dinv_ref[...] = dinv


_tc_first = pl.pallas_call(
    _tc_first_body,
    grid=(GB,),
    in_specs=[
        pl.BlockSpec((NC, BN, DEG_W), lambda i: (0, i, 0)),
        pl.BlockSpec((BN, D_IN), lambda i: (i, 0)),
        pl.BlockSpec((D_IN, D_H), lambda i: (0, 0)),
    ],
    out_specs=(
        pl.BlockSpec((BN, D_H), lambda i: (i, 0)),
        pl.BlockSpec((BN, 1), lambda i: (i, 0)),
    ),
    out_shape=(
        jax.ShapeDtypeStruct((N, D_H), jnp.bfloat16),
        jax.ShapeDtypeStruct((N, 1), jnp.float32),
    ),
)


def _tc_mid_body(parts_ref, hp_ref, dinv_ref, b_ref, w_ref, hpn_ref):
    dinv = dinv_ref[...]
    agg = (parts_ref[0].astype(jnp.float32)
           + parts_ref[1].astype(jnp.float32)
           + hp_ref[...].astype(jnp.float32))
    xl = _leaky(agg * dinv + b_ref[...])
    h = jnp.dot(xl, w_ref[...], preferred_element_type=jnp.float32)
    hpn_ref[...] = (h * dinv).astype(jnp.bfloat16)


_tc_mid = pl.pallas_call(
    _tc_mid_body,
    grid=(GB,),
    in_specs=[
        pl.BlockSpec((NC, BN, D_H), lambda i: (0, i, 0)),
        pl.BlockSpec((BN, D_H), lambda i: (i, 0)),
        pl.BlockSpec((BN, 1), lambda i: (i, 0)),
        pl.BlockSpec((1, D_H), lambda i: (0, 0)),
        pl.BlockSpec((D_H, D_H), lambda i: (0, 0)),
    ],
    out_specs=pl.BlockSpec((BN, D_H), lambda i: (i, 0)),
    out_shape=jax.ShapeDtypeStruct((N, D_H), jnp.bfloat16),
)


def _tc_final_body(parts_ref, hp_ref, dinv_ref, b_ref, fc1w_ref, fc1b_ref,
                   fc2w_ref, fc2b_ref, out_ref, acc_ref):
    i = pl.program_id(0)
    dinv = dinv_ref[...]
    agg = (parts_ref[0].astype(jnp.float32)
           + parts_ref[1].astype(jnp.float32)
           + hp_ref[...].astype(jnp.float32))
    xl = _leaky(agg * dinv + b_ref[...])
    colsum = jnp.sum(xl, axis=0, keepdims=True)

    @pl.when(i == 0)
    def _():
        acc_ref[...] = colsum

    @pl.when(i > 0)
    def _():
        acc_ref[...] += colsum

    @pl.when(i == GB - 1)
    def _():
        g = acc_ref[...] * (1.0 / N)
        z = _leaky(
            jnp.dot(g, fc1w_ref[...], preferred_element_type=jnp.float32)
            + fc1b_ref[...]
        )
        z = (
            jnp.dot(z, fc2w_ref[...], preferred_element_type=jnp.float32)
            + fc2b_ref[...]
        )
        out_ref[...] = 1.0 / (1.0 + jnp.exp(-z))


_tc_final = pl.pallas_call(
    _tc_final_body,
    grid=(GB,),
    in_specs=[
        pl.BlockSpec((NC, BN, D_H), lambda i: (0, i, 0)),
        pl.BlockSpec((BN, D_H), lambda i: (i, 0)),
        pl.BlockSpec((BN, 1), lambda i: (i, 0)),
        pl.BlockSpec((1, D_H), lambda i: (0, 0)),
        pl.BlockSpec((D_H, D_H), lambda i: (0, 0)),
        pl.BlockSpec((1, D_H), lambda i: (0, 0)),
        pl.BlockSpec((D_H, 1), lambda i: (0, 0)),
        pl.BlockSpec((1, 1), lambda i: (0, 0)),
    ],
    out_specs=pl.BlockSpec((1, 1), lambda i: (0, 0)),
    out_shape=jax.ShapeDtypeStruct((1, 1), jnp.float32),
    scratch_shapes=[pltpu.VMEM((1, D_H), jnp.float32)],
)


def kernel(x, edge_index, W1, b1, W2, b2, W3, b3, W4, b4,
           fc1_W, fc1_b, fc2_W, fc2_b):
    zeros_deg = jnp.zeros((ROWS_PER_TILE, DEG_W), jnp.float32)
    ones_deg = jnp.ones((CHUNK, DEG_W), jnp.float32)
    zeros_h = jnp.zeros((CHUNK, D_H), jnp.bfloat16)

    degp = _sc_deg(edge_index, zeros_deg, ones_deg)
    hp, dinv = _tc_first(degp, x, W1)
    for b, w in ((b1, W2), (b2, W3), (b3, W4)):
        parts = _sc_scat(hp, edge_index, zeros_h)
        hp = _tc_mid(parts, hp, dinv, b.reshape(1, D_H), w)
    parts = _sc_scat(hp, edge_index, zeros_h)
    return _tc_final(
        parts, hp, dinv, b4.reshape(1, D_H),
        fc1_W, fc1_b.reshape(1, D_H), fc2_W, fc2_b.reshape(1, 1),
    )


# speedup vs baseline: 1.0386x; 1.0386x over previous
import jax
import jax.numpy as jnp
from jax import lax
from jax.experimental import pallas as pl
from jax.experimental.pallas import tpu as pltpu
from jax.experimental.pallas import tpu_sc as plsc

N = 10000
E = 320000
D_IN = 128
D_H = 64
CHUNK = 128
NCHUNKS = E // CHUNK
NC = 2
NS = 16
NW = NC * NS
NPAD = 10240
ROWS_PER_TILE = NPAD // NS
CPT = NCHUNKS // NW
CPT_REM = NCHUNKS % NW
MAXCPT = CPT + 1
NBUF = 8
TOUT = (MAXCPT + NBUF - 1) // NBUF
GB = 10
BN = NPAD // GB

_mesh = plsc.VectorSubcoreMesh(
    core_axis_name="c", subcore_axis_name="s", num_cores=NC, num_subcores=NS
)
_sc_params = pltpu.CompilerParams(use_tc_tiling_on_sc=False)
_sc_params_nolayout = pltpu.CompilerParams(
    use_tc_tiling_on_sc=False, needs_layout_passes=False
)


def _leaky(v):
    return jnp.where(v >= 0, v, 0.01 * v)


def _tile_chunk_range(w):
    count = jnp.where(w < CPT_REM, CPT + 1, CPT)
    start = CPT * w + jnp.minimum(w, CPT_REM)
    return start, count


def _preload_idx(ei, row, start, count, idx_all):
    base = pl.multiple_of(start * CHUNK, CHUNK)
    pltpu.sync_copy(ei.at[row, pl.ds(base, CPT * CHUNK)],
                    idx_all.at[pl.ds(0, CPT * CHUNK)])

    @pl.when(count > CPT)
    def _():
        base2 = pl.multiple_of((start + CPT) * CHUNK, CHUNK)
        pltpu.sync_copy(ei.at[row, pl.ds(base2, CHUNK)],
                        idx_all.at[pl.ds(CPT * CHUNK, CHUNK)])


def _chunk_idx(idx_all, j):
    return idx_all.at[pl.ds(pl.multiple_of(j * CHUNK, CHUNK), CHUNK)]


DEG_W = 16


def _sc_deg_body(ei, zeros_hbm, ones_hbm, out, didx_all, ones_v, zbuf, cbuf,
                 acc, ssem):
    c = lax.axis_index("c")
    s = lax.axis_index("s")
    w = s * NC + c
    start, count = _tile_chunk_range(w)

    pltpu.sync_copy(ones_hbm, ones_v)
    pltpu.sync_copy(zeros_hbm, zbuf)
    pltpu.sync_copy(zbuf, acc.at[pl.ds(s * ROWS_PER_TILE, ROWS_PER_TILE)])
    _preload_idx(ei, 1, start, count, didx_all)
    plsc.subcore_barrier()

    def obody(k, carry):
        base = k * NBUF
        for b in range(NBUF):
            j = base + b

            @pl.when(jnp.logical_and(k > 0, j < count))
            def _(b=b, j=j):
                pltpu.make_async_copy(
                    ones_v, acc.at[_chunk_idx(didx_all, j - NBUF)], ssem.at[b]
                ).wait()

            @pl.when(j < count)
            def _(b=b, j=j):
                pltpu.async_copy(
                    ones_v, acc.at[_chunk_idx(didx_all, j)], ssem.at[b],
                    add=True,
                )

        return carry

    lax.fori_loop(0, TOUT, obody, 0)
    for b in range(NBUF):
        pltpu.make_async_copy(
            ones_v, acc.at[_chunk_idx(didx_all, b)], ssem.at[b]
        ).wait()
    plsc.subcore_barrier()
    pltpu.sync_copy(
        acc.at[pl.ds(s * ROWS_PER_TILE, ROWS_PER_TILE)], zbuf
    )
    col0 = jnp.zeros((16,), jnp.int32)
    rowi = lax.iota(jnp.int32, 16)

    def cmpbody(r, carry):
        vals = plsc.load_gather(zbuf, [rowi + r * 16, col0])
        cbuf[pl.ds(r * 16, 16)] = vals
        return carry

    lax.fori_loop(0, ROWS_PER_TILE // 16, cmpbody, 0)
    pltpu.sync_copy(cbuf, out.at[c, pl.ds(s * ROWS_PER_TILE, ROWS_PER_TILE)])


_sc_deg = pl.kernel(
    _sc_deg_body,
    out_type=jax.ShapeDtypeStruct((NC, NPAD), jnp.float32),
    mesh=_mesh,
    compiler_params=_sc_params_nolayout,
    scratch_types=[
        pltpu.VMEM((MAXCPT * CHUNK,), jnp.int32),
        pltpu.VMEM((CHUNK, DEG_W), jnp.float32),
        pltpu.VMEM((ROWS_PER_TILE, DEG_W), jnp.float32),
        pltpu.VMEM((ROWS_PER_TILE,), jnp.float32),
        pltpu.VMEM_SHARED((NPAD, DEG_W), jnp.float32),
        pltpu.SemaphoreType.DMA((NBUF,)),
    ],
)


def _sc_scat_body(hp, ei, zeros_hbm, out, sidx_all, didx_all, rows,
                  acc, gsem, ssem):
    c = lax.axis_index("c")
    s = lax.axis_index("s")
    w = s * NC + c
    start, count = _tile_chunk_range(w)

    pltpu.sync_copy(zeros_hbm, rows.at[0])
    for i in range(ROWS_PER_TILE // CHUNK):
        pltpu.sync_copy(
            rows.at[0], acc.at[pl.ds(s * ROWS_PER_TILE + i * CHUNK, CHUNK)]
        )
    _preload_idx(ei, 0, start, count, sidx_all)
    _preload_idx(ei, 1, start, count, didx_all)
    plsc.subcore_barrier()

    def obody(k, carry):
        base = k * NBUF
        for b in range(NBUF):
            j = base + b

            @pl.when(jnp.logical_and(k > 0, j < count))
            def _(b=b, j=j):
                pltpu.make_async_copy(
                    rows.at[b], acc.at[_chunk_idx(didx_all, j - NBUF)],
                    ssem.at[b],
                ).wait()

            @pl.when(j < count)
            def _(b=b, j=j):
                pltpu.async_copy(
                    hp.at[_chunk_idx(sidx_all, j)], rows.at[b], gsem.at[b]
                )

        for b in range(NBUF):
            j = base + b

            @pl.when(j < count)
            def _(b=b, j=j):
                pltpu.make_async_copy(
                    hp.at[_chunk_idx(sidx_all, j)], rows.at[b], gsem.at[b]
                ).wait()
                pltpu.async_copy(
                    rows.at[b], acc.at[_chunk_idx(didx_all, j)], ssem.at[b],
                    add=True,
                )

        return carry

    lax.fori_loop(0, TOUT, obody, 0)
    for b in range(NBUF):
        pltpu.make_async_copy(
            rows.at[b], acc.at[_chunk_idx(didx_all, b)], ssem.at[b]
        ).wait()
    plsc.subcore_barrier()
    pltpu.sync_copy(
        acc.at[pl.ds(s * ROWS_PER_TILE, ROWS_PER_TILE)],
        out.at[c, pl.ds(s * ROWS_PER_TILE, ROWS_PER_TILE)],
    )


_sc_scat = pl.kernel(
    _sc_scat_body,
    out_type=jax.ShapeDtypeStruct((NC, NPAD, D_H), jnp.bfloat16),
    mesh=_mesh,
    compiler_params=_sc_params,
    scratch_types=[
        pltpu.VMEM((MAXCPT * CHUNK,), jnp.int32),
        pltpu.VMEM((MAXCPT * CHUNK,), jnp.int32),
        pltpu.VMEM((NBUF, CHUNK, D_H), jnp.bfloat16),
        pltpu.VMEM_SHARED((NPAD, D_H), jnp.bfloat16),
        pltpu.SemaphoreType.DMA((NBUF,)),
        pltpu.SemaphoreType.DMA((NBUF,)),
    ],
)


def _tc_first_body(degp_ref, x_ref, w1_ref, hp_ref, dinv_ref):
    deg = degp_ref[0] + degp_ref[1] + 1.0
    dinv1 = lax.rsqrt(deg)
    dinv = dinv1.reshape(BN, 1)
    h = jnp.dot(x_ref[...], w1_ref[...], preferred_element_type=jnp.float32)
    hp_ref[...] = (h * dinv).astype(jnp.bfloat16)
    dinv_ref[...] = dinv1


_tc_first = pl.pallas_call(
    _tc_first_body,
    grid=(GB,),
    in_specs=[
        pl.BlockSpec((NC, BN), lambda i: (0, i)),
        pl.BlockSpec((BN, D_IN), lambda i: (i, 0)),
        pl.BlockSpec((D_IN, D_H), lambda i: (0, 0)),
    ],
    out_specs=(
        pl.BlockSpec((BN, D_H), lambda i: (i, 0)),
        pl.BlockSpec((BN,), lambda i: (i,)),
    ),
    out_shape=(
        jax.ShapeDtypeStruct((NPAD, D_H), jnp.bfloat16),
        jax.ShapeDtypeStruct((NPAD,), jnp.float32),
    ),
)


def _tc_mid_body(parts_ref, hp_ref, dinv_ref, b_ref, w_ref, hpn_ref):
    dinv = dinv_ref[...].reshape(BN, 1)
    agg = (parts_ref[0].astype(jnp.float32)
           + parts_ref[1].astype(jnp.float32)
           + hp_ref[...].astype(jnp.float32))
    xl = _leaky(agg * dinv + b_ref[...])
    h = jnp.dot(xl, w_ref[...], preferred_element_type=jnp.float32)
    hpn_ref[...] = (h * dinv).astype(jnp.bfloat16)


_tc_mid = pl.pallas_call(
    _tc_mid_body,
    grid=(GB,),
    in_specs=[
        pl.BlockSpec((NC, BN, D_H), lambda i: (0, i, 0)),
        pl.BlockSpec((BN, D_H), lambda i: (i, 0)),
        pl.BlockSpec((BN,), lambda i: (i,)),
        pl.BlockSpec((1, D_H), lambda i: (0, 0)),
        pl.BlockSpec((D_H, D_H), lambda i: (0, 0)),
    ],
    out_specs=pl.BlockSpec((BN, D_H), lambda i: (i, 0)),
    out_shape=jax.ShapeDtypeStruct((NPAD, D_H), jnp.bfloat16),
)


def _tc_final_body(parts_ref, hp_ref, dinv_ref, b_ref, fc1w_ref, fc1b_ref,
                   fc2w_ref, fc2b_ref, out_ref, acc_ref):
    i = pl.program_id(0)
    dinv = dinv_ref[...].reshape(BN, 1)
    agg = (parts_ref[0].astype(jnp.float32)
           + parts_ref[1].astype(jnp.float32)
           + hp_ref[...].astype(jnp.float32))
    xl = _leaky(agg * dinv + b_ref[...])
    rows = i * BN + lax.broadcasted_iota(jnp.int32, (BN, 1), 0)
    xl = jnp.where(rows < N, xl, 0.0)
    colsum = jnp.sum(xl, axis=0, keepdims=True)

    @pl.when(i == 0)
    def _():
        acc_ref[...] = colsum

    @pl.when(i > 0)
    def _():
        acc_ref[...] += colsum

    @pl.when(i == GB - 1)
    def _():
        g = acc_ref[...] * (1.0 / N)
        z = _leaky(
            jnp.dot(g, fc1w_ref[...], preferred_element_type=jnp.float32)
            + fc1b_ref[...]
        )
        z = (
            jnp.dot(z, fc2w_ref[...], preferred_element_type=jnp.float32)
            + fc2b_ref[...]
        )
        out_ref[...] = 1.0 / (1.0 + jnp.exp(-z))


_tc_final = pl.pallas_call(
    _tc_final_body,
    grid=(GB,),
    in_specs=[
        pl.BlockSpec((NC, BN, D_H), lambda i: (0, i, 0)),
        pl.BlockSpec((BN, D_H), lambda i: (i, 0)),
        pl.BlockSpec((BN,), lambda i: (i,)),
        pl.BlockSpec((1, D_H), lambda i: (0, 0)),
        pl.BlockSpec((D_H, D_H), lambda i: (0, 0)),
        pl.BlockSpec((1, D_H), lambda i: (0, 0)),
        pl.BlockSpec((D_H, 1), lambda i: (0, 0)),
        pl.BlockSpec((1, 1), lambda i: (0, 0)),
    ],
    out_specs=pl.BlockSpec((1, 1), lambda i: (0, 0)),
    out_shape=jax.ShapeDtypeStruct((1, 1), jnp.float32),
    scratch_shapes=[pltpu.VMEM((1, D_H), jnp.float32)],
)


def kernel(x, edge_index, W1, b1, W2, b2, W3, b3, W4, b4,
           fc1_W, fc1_b, fc2_W, fc2_b):
    zeros_deg = jnp.zeros((ROWS_PER_TILE, DEG_W), jnp.float32)
    ones_deg = jnp.ones((CHUNK, DEG_W), jnp.float32)
    zeros_h = jnp.zeros((CHUNK, D_H), jnp.bfloat16)

    degp = _sc_deg(edge_index, zeros_deg, ones_deg)
    hp, dinv = _tc_first(degp, x, W1)
    for b, w in ((b1, W2), (b2, W3), (b3, W4)):
        parts = _sc_scat(hp, edge_index, zeros_h)
        hp = _tc_mid(parts, hp, dinv, b.reshape(1, D_H), w)
    parts = _sc_scat(hp, edge_index, zeros_h)
    return _tc_final(
        parts, hp, dinv, b4.reshape(1, D_H),
        fc1_W, fc1_b.reshape(1, D_H), fc2_W, fc2_b.reshape(1, 1),
    )


# speedup vs baseline: 1.0754x; 1.0354x over previous
import jax
import jax.numpy as jnp
from jax import lax
from jax.experimental import pallas as pl
from jax.experimental.pallas import tpu as pltpu
from jax.experimental.pallas import tpu_sc as plsc

N = 10000
E = 320000
D_IN = 128
D_H = 64
CHUNK = 128
NCHUNKS = E // CHUNK
NC = 2
NS = 16
NW = NC * NS
NPAD = 10240
ROWS_PER_TILE = NPAD // NS
CPT = NCHUNKS // NW
CPT_REM = NCHUNKS % NW
MAXCPT = CPT + 1
NBUF = 8
TOUT = (MAXCPT + NBUF - 1) // NBUF
GB = 5
BN = NPAD // GB

_mesh = plsc.VectorSubcoreMesh(
    core_axis_name="c", subcore_axis_name="s", num_cores=NC, num_subcores=NS
)
_sc_params = pltpu.CompilerParams(use_tc_tiling_on_sc=False)
_sc_params_nolayout = pltpu.CompilerParams(
    use_tc_tiling_on_sc=False, needs_layout_passes=False
)


def _leaky(v):
    return jnp.where(v >= 0, v, 0.01 * v)


def _tile_chunk_range(w):
    count = jnp.where(w < CPT_REM, CPT + 1, CPT)
    start = CPT * w + jnp.minimum(w, CPT_REM)
    return start, count


def _preload_idx(ei, row, start, count, idx_all):
    base = pl.multiple_of(start * CHUNK, CHUNK)
    pltpu.sync_copy(ei.at[row, pl.ds(base, CPT * CHUNK)],
                    idx_all.at[pl.ds(0, CPT * CHUNK)])

    @pl.when(count > CPT)
    def _():
        base2 = pl.multiple_of((start + CPT) * CHUNK, CHUNK)
        pltpu.sync_copy(ei.at[row, pl.ds(base2, CHUNK)],
                        idx_all.at[pl.ds(CPT * CHUNK, CHUNK)])


def _chunk_idx(idx_all, j):
    return idx_all.at[pl.ds(pl.multiple_of(j * CHUNK, CHUNK), CHUNK)]


DEG_W = 16


def _sc_deg_body(ei, zeros_hbm, ones_hbm, out, didx_all, ones_v, zbuf, cbuf,
                 acc, ssem):
    c = lax.axis_index("c")
    s = lax.axis_index("s")
    w = s * NC + c
    start, count = _tile_chunk_range(w)

    pltpu.sync_copy(ones_hbm, ones_v)
    pltpu.sync_copy(zeros_hbm, zbuf)
    pltpu.sync_copy(zbuf, acc.at[pl.ds(s * ROWS_PER_TILE, ROWS_PER_TILE)])
    _preload_idx(ei, 1, start, count, didx_all)
    plsc.subcore_barrier()

    def obody(k, carry):
        base = k * NBUF
        for b in range(NBUF):
            j = base + b

            @pl.when(jnp.logical_and(k > 0, j < count))
            def _(b=b, j=j):
                pltpu.make_async_copy(
                    ones_v, acc.at[_chunk_idx(didx_all, j - NBUF)], ssem.at[b]
                ).wait()

            @pl.when(j < count)
            def _(b=b, j=j):
                pltpu.async_copy(
                    ones_v, acc.at[_chunk_idx(didx_all, j)], ssem.at[b],
                    add=True,
                )

        return carry

    lax.fori_loop(0, TOUT, obody, 0)
    for b in range(NBUF):
        pltpu.make_async_copy(
            ones_v, acc.at[_chunk_idx(didx_all, b)], ssem.at[b]
        ).wait()
    plsc.subcore_barrier()
    pltpu.sync_copy(
        acc.at[pl.ds(s * ROWS_PER_TILE, ROWS_PER_TILE)], zbuf
    )
    col0 = jnp.zeros((16,), jnp.int32)
    rowi = lax.iota(jnp.int32, 16)

    def cmpbody(r, carry):
        vals = plsc.load_gather(zbuf, [rowi + r * 16, col0])
        cbuf[pl.ds(r * 16, 16)] = vals
        return carry

    lax.fori_loop(0, ROWS_PER_TILE // 16, cmpbody, 0)
    pltpu.sync_copy(cbuf, out.at[c, pl.ds(s * ROWS_PER_TILE, ROWS_PER_TILE)])


_sc_deg = pl.kernel(
    _sc_deg_body,
    out_type=jax.ShapeDtypeStruct((NC, NPAD), jnp.float32),
    mesh=_mesh,
    compiler_params=_sc_params_nolayout,
    scratch_types=[
        pltpu.VMEM((MAXCPT * CHUNK,), jnp.int32),
        pltpu.VMEM((CHUNK, DEG_W), jnp.float32),
        pltpu.VMEM((ROWS_PER_TILE, DEG_W), jnp.float32),
        pltpu.VMEM((ROWS_PER_TILE,), jnp.float32),
        pltpu.VMEM_SHARED((NPAD, DEG_W), jnp.float32),
        pltpu.SemaphoreType.DMA((NBUF,)),
    ],
)


def _sc_scat_body(hp, ei, zeros_hbm, out, sidx_all, didx_all, rows,
                  acc, gsem, ssem):
    c = lax.axis_index("c")
    s = lax.axis_index("s")
    w = s * NC + c
    start, count = _tile_chunk_range(w)

    pltpu.sync_copy(zeros_hbm, rows.at[0])
    for i in range(ROWS_PER_TILE // CHUNK):
        pltpu.sync_copy(
            rows.at[0], acc.at[pl.ds(s * ROWS_PER_TILE + i * CHUNK, CHUNK)]
        )
    _preload_idx(ei, 0, start, count, sidx_all)
    _preload_idx(ei, 1, start, count, didx_all)
    plsc.subcore_barrier()

    def obody(k, carry):
        base = k * NBUF
        for b in range(NBUF):
            j = base + b

            @pl.when(jnp.logical_and(k > 0, j < count))
            def _(b=b, j=j):
                pltpu.make_async_copy(
                    rows.at[b], acc.at[_chunk_idx(didx_all, j - NBUF)],
                    ssem.at[b],
                ).wait()

            @pl.when(j < count)
            def _(b=b, j=j):
                pltpu.async_copy(
                    hp.at[_chunk_idx(sidx_all, j)], rows.at[b], gsem.at[b]
                )

        for b in range(NBUF):
            j = base + b

            @pl.when(j < count)
            def _(b=b, j=j):
                pltpu.make_async_copy(
                    hp.at[_chunk_idx(sidx_all, j)], rows.at[b], gsem.at[b]
                ).wait()
                pltpu.async_copy(
                    rows.at[b], acc.at[_chunk_idx(didx_all, j)], ssem.at[b],
                    add=True,
                )

        return carry

    lax.fori_loop(0, TOUT, obody, 0)
    for b in range(NBUF):
        pltpu.make_async_copy(
            rows.at[b], acc.at[_chunk_idx(didx_all, b)], ssem.at[b]
        ).wait()
    plsc.subcore_barrier()
    pltpu.sync_copy(
        acc.at[pl.ds(s * ROWS_PER_TILE, ROWS_PER_TILE)],
        out.at[c, pl.ds(s * ROWS_PER_TILE, ROWS_PER_TILE)],
    )


_sc_scat = pl.kernel(
    _sc_scat_body,
    out_type=jax.ShapeDtypeStruct((NC, NPAD, D_H), jnp.bfloat16),
    mesh=_mesh,
    compiler_params=_sc_params,
    scratch_types=[
        pltpu.VMEM((MAXCPT * CHUNK,), jnp.int32),
        pltpu.VMEM((MAXCPT * CHUNK,), jnp.int32),
        pltpu.VMEM((NBUF, CHUNK, D_H), jnp.bfloat16),
        pltpu.VMEM_SHARED((NPAD, D_H), jnp.bfloat16),
        pltpu.SemaphoreType.DMA((NBUF,)),
        pltpu.SemaphoreType.DMA((NBUF,)),
    ],
)


def _tc_first_body(degp_ref, x_ref, w1_ref, hp_ref, dinv_ref):
    deg = degp_ref[0] + degp_ref[1] + 1.0
    dinv1 = lax.rsqrt(deg)
    dinv = dinv1.reshape(BN, 1)
    h = jnp.dot(x_ref[...], w1_ref[...], preferred_element_type=jnp.float32)
    hp_ref[...] = (h * dinv).astype(jnp.bfloat16)
    dinv_ref[...] = dinv1


_tc_first = pl.pallas_call(
    _tc_first_body,
    grid=(GB,),
    in_specs=[
        pl.BlockSpec((NC, BN), lambda i: (0, i)),
        pl.BlockSpec((BN, D_IN), lambda i: (i, 0)),
        pl.BlockSpec((D_IN, D_H), lambda i: (0, 0)),
    ],
    out_specs=(
        pl.BlockSpec((BN, D_H), lambda i: (i, 0)),
        pl.BlockSpec((BN,), lambda i: (i,)),
    ),
    out_shape=(
        jax.ShapeDtypeStruct((NPAD, D_H), jnp.bfloat16),
        jax.ShapeDtypeStruct((NPAD,), jnp.float32),
    ),
)


def _tc_mid_body(parts_ref, hp_ref, dinv_ref, b_ref, w_ref, hpn_ref):
    dinv = dinv_ref[...].reshape(BN, 1)
    agg = (parts_ref[0].astype(jnp.float32)
           + parts_ref[1].astype(jnp.float32)
           + hp_ref[...].astype(jnp.float32))
    xl = _leaky(agg * dinv + b_ref[...])
    h = jnp.dot(xl, w_ref[...], preferred_element_type=jnp.float32)
    hpn_ref[...] = (h * dinv).astype(jnp.bfloat16)


_tc_mid = pl.pallas_call(
    _tc_mid_body,
    grid=(GB,),
    in_specs=[
        pl.BlockSpec((NC, BN, D_H), lambda i: (0, i, 0)),
        pl.BlockSpec((BN, D_H), lambda i: (i, 0)),
        pl.BlockSpec((BN,), lambda i: (i,)),
        pl.BlockSpec((1, D_H), lambda i: (0, 0)),
        pl.BlockSpec((D_H, D_H), lambda i: (0, 0)),
    ],
    out_specs=pl.BlockSpec((BN, D_H), lambda i: (i, 0)),
    out_shape=jax.ShapeDtypeStruct((NPAD, D_H), jnp.bfloat16),
)


def _tc_final_body(parts_ref, hp_ref, dinv_ref, b_ref, fc1w_ref, fc1b_ref,
                   fc2w_ref, fc2b_ref, out_ref, acc_ref):
    i = pl.program_id(0)
    dinv = dinv_ref[...].reshape(BN, 1)
    agg = (parts_ref[0].astype(jnp.float32)
           + parts_ref[1].astype(jnp.float32)
           + hp_ref[...].astype(jnp.float32))
    xl = _leaky(agg * dinv + b_ref[...])
    rows = i * BN + lax.broadcasted_iota(jnp.int32, (BN, 1), 0)
    xl = jnp.where(rows < N, xl, 0.0)
    colsum = jnp.sum(xl, axis=0, keepdims=True)

    @pl.when(i == 0)
    def _():
        acc_ref[...] = colsum

    @pl.when(i > 0)
    def _():
        acc_ref[...] += colsum

    @pl.when(i == GB - 1)
    def _():
        g = acc_ref[...] * (1.0 / N)
        z = _leaky(
            jnp.dot(g, fc1w_ref[...], preferred_element_type=jnp.float32)
            + fc1b_ref[...]
        )
        z = (
            jnp.dot(z, fc2w_ref[...], preferred_element_type=jnp.float32)
            + fc2b_ref[...]
        )
        out_ref[...] = 1.0 / (1.0 + jnp.exp(-z))


_tc_final = pl.pallas_call(
    _tc_final_body,
    grid=(GB,),
    in_specs=[
        pl.BlockSpec((NC, BN, D_H), lambda i: (0, i, 0)),
        pl.BlockSpec((BN, D_H), lambda i: (i, 0)),
        pl.BlockSpec((BN,), lambda i: (i,)),
        pl.BlockSpec((1, D_H), lambda i: (0, 0)),
        pl.BlockSpec((D_H, D_H), lambda i: (0, 0)),
        pl.BlockSpec((1, D_H), lambda i: (0, 0)),
        pl.BlockSpec((D_H, 1), lambda i: (0, 0)),
        pl.BlockSpec((1, 1), lambda i: (0, 0)),
    ],
    out_specs=pl.BlockSpec((1, 1), lambda i: (0, 0)),
    out_shape=jax.ShapeDtypeStruct((1, 1), jnp.float32),
    scratch_shapes=[pltpu.VMEM((1, D_H), jnp.float32)],
)


def kernel(x, edge_index, W1, b1, W2, b2, W3, b3, W4, b4,
           fc1_W, fc1_b, fc2_W, fc2_b):
    zeros_deg = jnp.zeros((ROWS_PER_TILE, DEG_W), jnp.float32)
    ones_deg = jnp.ones((CHUNK, DEG_W), jnp.float32)
    zeros_h = jnp.zeros((CHUNK, D_H), jnp.bfloat16)

    degp = _sc_deg(edge_index, zeros_deg, ones_deg)
    hp, dinv = _tc_first(degp, x, W1)
    for b, w in ((b1, W2), (b2, W3), (b3, W4)):
        parts = _sc_scat(hp, edge_index, zeros_h)
        hp = _tc_mid(parts, hp, dinv, b.reshape(1, D_H), w)
    parts = _sc_scat(hp, edge_index, zeros_h)
    return _tc_final(
        parts, hp, dinv, b4.reshape(1, D_H),
        fc1_W, fc1_b.reshape(1, D_H), fc2_W, fc2_b.reshape(1, 1),
    )
